# baseline (speedup 1.0000x reference)
import functools

import jax
import jax.numpy as jnp
from jax import lax
from jax.experimental import pallas as pl
from jax.experimental.pallas import tpu as pltpu
from jax.experimental.pallas import tpu_sc as plsc

N = 10000
E = 320000
D = 128

NC = 2
NS = 16
NW = NC * NS
E_PER_W = E // NW
CHUNK = 50
N_CHUNKS = E_PER_W // CHUNK
KBLK = 40
NBLK = N_CHUNKS // KBLK
NB = 4
LOOK = 2
RPT = 624
TAIL = N - NS * RPT


def _make_sc_agg(compute_deg: bool):
  mesh = plsc.VectorSubcoreMesh(core_axis_name="c", subcore_axis_name="s")
  out_type = [jax.ShapeDtypeStruct((NC, N, D), jnp.float32)]
  if compute_deg:
    out_type.append(jax.ShapeDtypeStruct((NC, N), jnp.float32))

  @functools.partial(
      pl.kernel,
      mesh=mesh,
      out_type=tuple(out_type),
      scratch_types=(
          [pltpu.VMEM((KBLK, CHUNK), jnp.int32) for _ in range(4)]
          + [pltpu.VMEM((CHUNK, D), jnp.float32) for _ in range(NB)]
          + [
              pltpu.VMEM((CHUNK,), jnp.float32),
              pltpu.VMEM_SHARED((N, D), jnp.float32),
              pltpu.VMEM_SHARED((N,), jnp.float32),
          ]
          + [pltpu.SemaphoreType.DMA for _ in range(2 * NB + 2)]
      ),
  )
  def sc_agg(x_hbm, src_hbm, dst_hbm, zrow_hbm, zdeg_hbm, ones_hbm, *rest):
    if compute_deg:
      agg_out, deg_out = rest[0], rest[1]
      scratch = rest[2:]
    else:
      agg_out = rest[0]
      scratch = rest[1:]
    srcA, dstA, srcB, dstB = scratch[0:4]
    rows = scratch[4:4 + NB]
    ones_v, acc, deg_acc = scratch[4 + NB:7 + NB]
    gsem = scratch[7 + NB:7 + 2 * NB]
    ssem = scratch[7 + 2 * NB:7 + 3 * NB]
    dsem, isem = scratch[7 + 3 * NB:9 + 3 * NB]
    idxbufs = ((srcA, dstA), (srcB, dstB))

    c = lax.axis_index("c")
    s = lax.axis_index("s")
    wid = c * NS + s

    pltpu.sync_copy(zrow_hbm.at[pl.ds(s * RPT, RPT)],
                    acc.at[pl.ds(s * RPT, RPT)])
    @pl.when(s == NS - 1)
    def _():
      pltpu.sync_copy(zrow_hbm.at[pl.ds(NS * RPT, TAIL)],
                      acc.at[pl.ds(NS * RPT, TAIL)])
    if compute_deg:
      @pl.when(s == 0)
      def _():
        pltpu.sync_copy(zdeg_hbm, deg_acc)
      pltpu.sync_copy(ones_hbm, ones_v)
    plsc.subcore_barrier()

    def visit(i, b, src_v, dst_v, skip_drain, guard_issue):
      t = (b + LOOK) % NB
      pltpu.make_async_copy(x_hbm.at[src_v.at[i]], rows[b], gsem[b]).wait()
      if compute_deg:
        pltpu.async_copy(ones_v, deg_acc.at[dst_v.at[i]], dsem, add=True)
      pltpu.async_copy(rows[b], acc.at[dst_v.at[i]], ssem[b], add=True)
      if not skip_drain:
        pltpu.make_async_copy(rows[t], acc.at[dst_v.at[0]], ssem[t]).wait()
      if guard_issue:
        @pl.when(i + LOOK < KBLK)
        def _():
          pltpu.async_copy(x_hbm.at[src_v.at[i + LOOK]], rows[t], gsem[t])
      else:
        pltpu.async_copy(x_hbm.at[src_v.at[i + LOOK]], rows[t], gsem[t])

    for j in range(NBLK):
      src_v, dst_v = idxbufs[j % 2]
      if j == 0:
        pltpu.sync_copy(src_hbm.at[wid].at[0], src_v)
        pltpu.sync_copy(dst_hbm.at[wid].at[0], dst_v)
      else:
        pltpu.make_async_copy(src_hbm.at[wid].at[0], src_v, isem).wait()
        pltpu.make_async_copy(dst_hbm.at[wid].at[0], dst_v, isem).wait()
      for i in range(LOOK):
        pltpu.async_copy(x_hbm.at[src_v.at[i]], rows[i], gsem[i])

      for i in range(NB):
        visit(i, i, src_v, dst_v,
              skip_drain=(j == 0 and i < LOOK), guard_issue=False)

      if j + 1 < NBLK:
        nsrc, ndst = idxbufs[(j + 1) % 2]
        pltpu.async_copy(src_hbm.at[wid].at[j + 1], nsrc, isem)
        pltpu.async_copy(dst_hbm.at[wid].at[j + 1], ndst, isem)

      def body(o, carry2):
        for b4 in range(NB):
          visit(o * NB + b4, b4, src_v, dst_v,
                skip_drain=False, guard_issue=(b4 >= NB - LOOK))
        return carry2

      lax.fori_loop(1, KBLK // NB, body, 0)

    src_v, dst_v = idxbufs[(NBLK - 1) % 2]
    for i in range(KBLK - LOOK, KBLK):
      pltpu.make_async_copy(rows[i % NB], acc.at[dst_v.at[0]], ssem[i % NB]).wait()
    if compute_deg:
      def drain(i, carry):
        pltpu.make_async_copy(ones_v, deg_acc.at[dst_v.at[0]], dsem).wait()
        return carry
      lax.fori_loop(0, N_CHUNKS, drain, 0)
    plsc.subcore_barrier()

    pltpu.sync_copy(acc.at[pl.ds(s * RPT, RPT)],
                    agg_out.at[c].at[pl.ds(s * RPT, RPT)])
    @pl.when(s == NS - 1)
    def _():
      pltpu.sync_copy(acc.at[pl.ds(NS * RPT, TAIL)],
                      agg_out.at[c].at[pl.ds(NS * RPT, TAIL)])
    if compute_deg:
      @pl.when(s == 0)
      def _():
        pltpu.sync_copy(deg_acc, deg_out.at[c])

  return sc_agg


_sc_agg_deg = _make_sc_agg(True)
_sc_agg = _make_sc_agg(False)


BLK = 1000


def _tc_layer1(x_ref, agg_ref, deg_ref, wa_ref, ba_ref, wb_ref, bb_ref, o_ref):
  deg = deg_ref[0] + deg_ref[1]
  agg = agg_ref[0] + agg_ref[1]
  h = x_ref[...] + agg * (1.0 / jnp.maximum(deg, 1.0))
  t = jnp.maximum(jnp.dot(h, wa_ref[...], preferred_element_type=jnp.float32)
                  + ba_ref[...], 0.0)
  y = jnp.dot(t, wb_ref[...], preferred_element_type=jnp.float32) + bb_ref[...]
  o_ref[...] = jnp.maximum(y, 0.0)


def _tc_layer23(x_ref, agg_ref, deg_ref, w2a_ref, b2a_ref, w2b_ref, b2b_ref,
                w3a_ref, b3a_ref, w3b_ref, b3b_ref, o_ref):
  deg = deg_ref[0] + deg_ref[1]
  agg = agg_ref[0] + agg_ref[1]
  h = x_ref[...] + agg * (1.0 / jnp.maximum(deg, 1.0))
  t = jnp.maximum(jnp.dot(h, w2a_ref[...], preferred_element_type=jnp.float32)
                  + b2a_ref[...], 0.0)
  x2 = jnp.maximum(jnp.dot(t, w2b_ref[...], preferred_element_type=jnp.float32)
                   + b2b_ref[...], 0.0)
  t3 = jnp.maximum(jnp.dot(x2, w3a_ref[...], preferred_element_type=jnp.float32)
                   + b3a_ref[...], 0.0)
  o_ref[...] = (jnp.dot(t3, w3b_ref[...], preferred_element_type=jnp.float32)
                + b3b_ref[...])


def _row_spec():
  return pl.BlockSpec((BLK, D), lambda i: (i, 0))


def _agg_spec():
  return pl.BlockSpec((NC, BLK, D), lambda i: (0, i, 0))


def _deg_spec():
  return pl.BlockSpec((NC, BLK, 1), lambda i: (0, i, 0))


def _w_spec():
  return pl.BlockSpec((D, D), lambda i: (0, 0))


def _b_spec():
  return pl.BlockSpec((1, D), lambda i: (0, 0))


def kernel(features, edge_index, W1a, b1a, W1b, b1b, W2a, b2a, W2b, b2b,
           W3a, b3a, W3b, b3b):
  src = edge_index[0].reshape(NW, NBLK, KBLK, CHUNK)
  dst = edge_index[1].reshape(NW, NBLK, KBLK, CHUNK)
  zrow = jnp.zeros((N, D), jnp.float32)
  zdeg = jnp.zeros((N,), jnp.float32)
  ones = jnp.ones((CHUNK,), jnp.float32)

  aggp1, degp = _sc_agg_deg(features, src, dst, zrow, zdeg, ones)
  degp3 = degp.reshape(NC, N, 1)

  x1 = pl.pallas_call(
      _tc_layer1,
      grid=(N // BLK,),
      in_specs=[_row_spec(), _agg_spec(), _deg_spec(),
                _w_spec(), _b_spec(), _w_spec(), _b_spec()],
      out_specs=_row_spec(),
      out_shape=jax.ShapeDtypeStruct((N, D), jnp.float32),
  )(features, aggp1, degp3, W1a, b1a.reshape(1, D), W1b, b1b.reshape(1, D))

  (aggp2,) = _sc_agg(x1, src, dst, zrow, zdeg, ones)

  out = pl.pallas_call(
      _tc_layer23,
      grid=(N // BLK,),
      in_specs=[_row_spec(), _agg_spec(), _deg_spec(),
                _w_spec(), _b_spec(), _w_spec(), _b_spec(),
                _w_spec(), _b_spec(), _w_spec(), _b_spec()],
      out_specs=_row_spec(),
      out_shape=jax.ShapeDtypeStruct((N, D), jnp.float32),
  )(x1, aggp2, degp3, W2a, b2a.reshape(1, D), W2b, b2b.reshape(1, D),
    W3a, b3a.reshape(1, D), W3b, b3b.reshape(1, D))
  return out


# speedup vs baseline: 1.1536x; 1.1536x over previous
import functools

import jax
import jax.numpy as jnp
from jax import lax
from jax.experimental import pallas as pl
from jax.experimental.pallas import tpu as pltpu
from jax.experimental.pallas import tpu_sc as plsc

N = 10000
E = 320000
D = 128

NC = 2
NS = 16
NW = NC * NS
E_PER_W = E // NW
CHUNK = 50
N_CHUNKS = E_PER_W // CHUNK
KBLK = 40
NBLK = N_CHUNKS // KBLK
NB = 4
LOOK = 3
RPT = 624
TAIL = N - NS * RPT


def _make_sc_agg(compute_deg: bool):
  mesh = plsc.VectorSubcoreMesh(core_axis_name="c", subcore_axis_name="s")
  out_type = [jax.ShapeDtypeStruct((NC, N, D), jnp.float32)]
  if compute_deg:
    out_type.append(jax.ShapeDtypeStruct((NC, N), jnp.float32))

  @functools.partial(
      pl.kernel,
      mesh=mesh,
      out_type=tuple(out_type),
      scratch_types=(
          [pltpu.VMEM((KBLK, CHUNK), jnp.int32) for _ in range(4)]
          + [pltpu.VMEM((CHUNK, D), jnp.float32) for _ in range(NB)]
          + [
              pltpu.VMEM((CHUNK,), jnp.float32),
              pltpu.VMEM_SHARED((N, D), jnp.float32),
              pltpu.VMEM_SHARED((N,), jnp.float32),
          ]
          + [pltpu.SemaphoreType.DMA for _ in range(2 * NB + 2)]
      ),
  )
  def sc_agg(x_hbm, src_hbm, dst_hbm, zrow_hbm, zdeg_hbm, ones_hbm, *rest):
    if compute_deg:
      agg_out, deg_out = rest[0], rest[1]
      scratch = rest[2:]
    else:
      agg_out = rest[0]
      scratch = rest[1:]
    srcA, dstA, srcB, dstB = scratch[0:4]
    rows = scratch[4:4 + NB]
    ones_v, acc, deg_acc = scratch[4 + NB:7 + NB]
    gsem = scratch[7 + NB:7 + 2 * NB]
    ssem = scratch[7 + 2 * NB:7 + 3 * NB]
    dsem, isem = scratch[7 + 3 * NB:9 + 3 * NB]
    idxbufs = ((srcA, dstA), (srcB, dstB))

    c = lax.axis_index("c")
    s = lax.axis_index("s")
    wid = c * NS + s

    pltpu.sync_copy(zrow_hbm.at[pl.ds(s * RPT, RPT)],
                    acc.at[pl.ds(s * RPT, RPT)])
    @pl.when(s == NS - 1)
    def _():
      pltpu.sync_copy(zrow_hbm.at[pl.ds(NS * RPT, TAIL)],
                      acc.at[pl.ds(NS * RPT, TAIL)])
    if compute_deg:
      @pl.when(s == 0)
      def _():
        pltpu.sync_copy(zdeg_hbm, deg_acc)
      pltpu.sync_copy(ones_hbm, ones_v)
    plsc.subcore_barrier()

    def visit(i, b, src_v, dst_v, skip_drain, guard_issue):
      t = (b + LOOK) % NB
      pltpu.make_async_copy(x_hbm.at[src_v.at[i]], rows[b], gsem[b]).wait()
      if compute_deg:
        pltpu.async_copy(ones_v, deg_acc.at[dst_v.at[i]], dsem, add=True)
      pltpu.async_copy(rows[b], acc.at[dst_v.at[i]], ssem[b], add=True)
      if not skip_drain:
        pltpu.make_async_copy(rows[t], acc.at[dst_v.at[0]], ssem[t]).wait()
      if guard_issue:
        @pl.when(i + LOOK < KBLK)
        def _():
          pltpu.async_copy(x_hbm.at[src_v.at[i + LOOK]], rows[t], gsem[t])
      else:
        pltpu.async_copy(x_hbm.at[src_v.at[i + LOOK]], rows[t], gsem[t])

    for j in range(NBLK):
      src_v, dst_v = idxbufs[j % 2]
      if j == 0:
        pltpu.sync_copy(src_hbm.at[wid].at[0], src_v)
        pltpu.sync_copy(dst_hbm.at[wid].at[0], dst_v)
      else:
        pltpu.make_async_copy(src_hbm.at[wid].at[0], src_v, isem).wait()
        pltpu.make_async_copy(dst_hbm.at[wid].at[0], dst_v, isem).wait()
      for i in range(LOOK):
        pltpu.async_copy(x_hbm.at[src_v.at[i]], rows[i], gsem[i])

      for i in range(NB):
        visit(i, i, src_v, dst_v,
              skip_drain=(j == 0 and i < NB - LOOK), guard_issue=False)

      if j + 1 < NBLK:
        nsrc, ndst = idxbufs[(j + 1) % 2]
        pltpu.async_copy(src_hbm.at[wid].at[j + 1], nsrc, isem)
        pltpu.async_copy(dst_hbm.at[wid].at[j + 1], ndst, isem)

      def body(o, carry2):
        for b4 in range(NB):
          visit(o * NB + b4, b4, src_v, dst_v,
                skip_drain=False, guard_issue=(b4 >= NB - LOOK))
        return carry2

      lax.fori_loop(1, KBLK // NB, body, 0)

    src_v, dst_v = idxbufs[(NBLK - 1) % 2]
    for i in range(KBLK - (NB - LOOK), KBLK):
      pltpu.make_async_copy(rows[i % NB], acc.at[dst_v.at[0]], ssem[i % NB]).wait()
    if compute_deg:
      def drain(i, carry):
        pltpu.make_async_copy(ones_v, deg_acc.at[dst_v.at[0]], dsem).wait()
        return carry
      lax.fori_loop(0, N_CHUNKS, drain, 0)
    plsc.subcore_barrier()

    pltpu.sync_copy(acc.at[pl.ds(s * RPT, RPT)],
                    agg_out.at[c].at[pl.ds(s * RPT, RPT)])
    @pl.when(s == NS - 1)
    def _():
      pltpu.sync_copy(acc.at[pl.ds(NS * RPT, TAIL)],
                      agg_out.at[c].at[pl.ds(NS * RPT, TAIL)])
    if compute_deg:
      @pl.when(s == 0)
      def _():
        pltpu.sync_copy(deg_acc, deg_out.at[c])

  return sc_agg


_sc_agg_deg = _make_sc_agg(True)
_sc_agg = _make_sc_agg(False)


BLK = 1000


def _tc_layer1(x_ref, agg_ref, deg_ref, wa_ref, ba_ref, wb_ref, bb_ref, o_ref):
  deg = deg_ref[0] + deg_ref[1]
  agg = agg_ref[0] + agg_ref[1]
  h = x_ref[...] + agg * (1.0 / jnp.maximum(deg, 1.0))
  t = jnp.maximum(jnp.dot(h, wa_ref[...], preferred_element_type=jnp.float32)
                  + ba_ref[...], 0.0)
  y = jnp.dot(t, wb_ref[...], preferred_element_type=jnp.float32) + bb_ref[...]
  o_ref[...] = jnp.maximum(y, 0.0)


def _tc_layer23(x_ref, agg_ref, deg_ref, w2a_ref, b2a_ref, w2b_ref, b2b_ref,
                w3a_ref, b3a_ref, w3b_ref, b3b_ref, o_ref):
  deg = deg_ref[0] + deg_ref[1]
  agg = agg_ref[0] + agg_ref[1]
  h = x_ref[...] + agg * (1.0 / jnp.maximum(deg, 1.0))
  t = jnp.maximum(jnp.dot(h, w2a_ref[...], preferred_element_type=jnp.float32)
                  + b2a_ref[...], 0.0)
  x2 = jnp.maximum(jnp.dot(t, w2b_ref[...], preferred_element_type=jnp.float32)
                   + b2b_ref[...], 0.0)
  t3 = jnp.maximum(jnp.dot(x2, w3a_ref[...], preferred_element_type=jnp.float32)
                   + b3a_ref[...], 0.0)
  o_ref[...] = (jnp.dot(t3, w3b_ref[...], preferred_element_type=jnp.float32)
                + b3b_ref[...])


def _row_spec():
  return pl.BlockSpec((BLK, D), lambda i: (i, 0))


def _agg_spec():
  return pl.BlockSpec((NC, BLK, D), lambda i: (0, i, 0))


def _deg_spec():
  return pl.BlockSpec((NC, BLK, 1), lambda i: (0, i, 0))


def _w_spec():
  return pl.BlockSpec((D, D), lambda i: (0, 0))


def _b_spec():
  return pl.BlockSpec((1, D), lambda i: (0, 0))


def kernel(features, edge_index, W1a, b1a, W1b, b1b, W2a, b2a, W2b, b2b,
           W3a, b3a, W3b, b3b):
  src = edge_index[0].reshape(NW, NBLK, KBLK, CHUNK)
  dst = edge_index[1].reshape(NW, NBLK, KBLK, CHUNK)
  zrow = jnp.zeros((N, D), jnp.float32)
  zdeg = jnp.zeros((N,), jnp.float32)
  ones = jnp.ones((CHUNK,), jnp.float32)

  aggp1, degp = _sc_agg_deg(features, src, dst, zrow, zdeg, ones)
  degp3 = degp.reshape(NC, N, 1)

  x1 = pl.pallas_call(
      _tc_layer1,
      grid=(N // BLK,),
      in_specs=[_row_spec(), _agg_spec(), _deg_spec(),
                _w_spec(), _b_spec(), _w_spec(), _b_spec()],
      out_specs=_row_spec(),
      out_shape=jax.ShapeDtypeStruct((N, D), jnp.float32),
  )(features, aggp1, degp3, W1a, b1a.reshape(1, D), W1b, b1b.reshape(1, D))

  (aggp2,) = _sc_agg(x1, src, dst, zrow, zdeg, ones)

  out = pl.pallas_call(
      _tc_layer23,
      grid=(N // BLK,),
      in_specs=[_row_spec(), _agg_spec(), _deg_spec(),
                _w_spec(), _b_spec(), _w_spec(), _b_spec(),
                _w_spec(), _b_spec(), _w_spec(), _b_spec()],
      out_specs=_row_spec(),
      out_shape=jax.ShapeDtypeStruct((N, D), jnp.float32),
  )(x1, aggp2, degp3, W2a, b2a.reshape(1, D), W2b, b2b.reshape(1, D),
    W3a, b3a.reshape(1, D), W3b, b3b.reshape(1, D))
  return out


# speedup vs baseline: 1.1540x; 1.0004x over previous
import functools

import jax
import jax.numpy as jnp
from jax import lax
from jax.experimental import pallas as pl
from jax.experimental.pallas import tpu as pltpu
from jax.experimental.pallas import tpu_sc as plsc

N = 10000
E = 320000
D = 128

NC = 2
NS = 16
NW = NC * NS
E_PER_W = E // NW
CHUNK = 50
N_CHUNKS = E_PER_W // CHUNK
KBLK = 20
NBLK = N_CHUNKS // KBLK
NB = 5
LOOK = 4
RPT = 624
TAIL = N - NS * RPT


def _make_sc_agg(compute_deg: bool):
  mesh = plsc.VectorSubcoreMesh(core_axis_name="c", subcore_axis_name="s")
  out_type = [jax.ShapeDtypeStruct((NC, N, D), jnp.float32)]
  if compute_deg:
    out_type.append(jax.ShapeDtypeStruct((NC, N), jnp.float32))

  @functools.partial(
      pl.kernel,
      mesh=mesh,
      out_type=tuple(out_type),
      scratch_types=(
          [pltpu.VMEM((KBLK, CHUNK), jnp.int32) for _ in range(4)]
          + [pltpu.VMEM((CHUNK, D), jnp.float32) for _ in range(NB)]
          + [
              pltpu.VMEM((CHUNK,), jnp.float32),
              pltpu.VMEM_SHARED((N, D), jnp.float32),
              pltpu.VMEM_SHARED((N,), jnp.float32),
          ]
          + [pltpu.SemaphoreType.DMA for _ in range(2 * NB + 2)]
      ),
  )
  def sc_agg(x_hbm, src_hbm, dst_hbm, zrow_hbm, zdeg_hbm, ones_hbm, *rest):
    if compute_deg:
      agg_out, deg_out = rest[0], rest[1]
      scratch = rest[2:]
    else:
      agg_out = rest[0]
      scratch = rest[1:]
    srcA, dstA, srcB, dstB = scratch[0:4]
    rows = scratch[4:4 + NB]
    ones_v, acc, deg_acc = scratch[4 + NB:7 + NB]
    gsem = scratch[7 + NB:7 + 2 * NB]
    ssem = scratch[7 + 2 * NB:7 + 3 * NB]
    dsem, isem = scratch[7 + 3 * NB:9 + 3 * NB]
    idxbufs = ((srcA, dstA), (srcB, dstB))

    c = lax.axis_index("c")
    s = lax.axis_index("s")
    wid = c * NS + s

    pltpu.sync_copy(zrow_hbm.at[pl.ds(s * RPT, RPT)],
                    acc.at[pl.ds(s * RPT, RPT)])
    @pl.when(s == NS - 1)
    def _():
      pltpu.sync_copy(zrow_hbm.at[pl.ds(NS * RPT, TAIL)],
                      acc.at[pl.ds(NS * RPT, TAIL)])
    if compute_deg:
      @pl.when(s == 0)
      def _():
        pltpu.sync_copy(zdeg_hbm, deg_acc)
      pltpu.sync_copy(ones_hbm, ones_v)
    plsc.subcore_barrier()

    def visit(i, b, src_v, dst_v, skip_drain, guard_issue):
      t = (b + LOOK) % NB
      pltpu.make_async_copy(x_hbm.at[src_v.at[i]], rows[b], gsem[b]).wait()
      if compute_deg:
        pltpu.async_copy(ones_v, deg_acc.at[dst_v.at[i]], dsem, add=True)
      pltpu.async_copy(rows[b], acc.at[dst_v.at[i]], ssem[b], add=True)
      if not skip_drain:
        pltpu.make_async_copy(rows[t], acc.at[dst_v.at[0]], ssem[t]).wait()
      if guard_issue:
        @pl.when(i + LOOK < KBLK)
        def _():
          pltpu.async_copy(x_hbm.at[src_v.at[i + LOOK]], rows[t], gsem[t])
      else:
        pltpu.async_copy(x_hbm.at[src_v.at[i + LOOK]], rows[t], gsem[t])

    for j in range(NBLK):
      src_v, dst_v = idxbufs[j % 2]
      if j == 0:
        pltpu.sync_copy(src_hbm.at[wid].at[0], src_v)
        pltpu.sync_copy(dst_hbm.at[wid].at[0], dst_v)
      else:
        pltpu.make_async_copy(src_hbm.at[wid].at[0], src_v, isem).wait()
        pltpu.make_async_copy(dst_hbm.at[wid].at[0], dst_v, isem).wait()
      for i in range(LOOK):
        pltpu.async_copy(x_hbm.at[src_v.at[i]], rows[i], gsem[i])

      for i in range(NB):
        visit(i, i, src_v, dst_v,
              skip_drain=(j == 0 and i < NB - LOOK), guard_issue=False)

      if j + 1 < NBLK:
        nsrc, ndst = idxbufs[(j + 1) % 2]
        pltpu.async_copy(src_hbm.at[wid].at[j + 1], nsrc, isem)
        pltpu.async_copy(dst_hbm.at[wid].at[j + 1], ndst, isem)

      def body(o, carry2):
        for b4 in range(NB):
          visit(o * NB + b4, b4, src_v, dst_v,
                skip_drain=False, guard_issue=(b4 >= NB - LOOK))
        return carry2

      lax.fori_loop(1, KBLK // NB, body, 0)

    src_v, dst_v = idxbufs[(NBLK - 1) % 2]
    for i in range(KBLK - (NB - LOOK), KBLK):
      pltpu.make_async_copy(rows[i % NB], acc.at[dst_v.at[0]], ssem[i % NB]).wait()
    if compute_deg:
      def drain(i, carry):
        pltpu.make_async_copy(ones_v, deg_acc.at[dst_v.at[0]], dsem).wait()
        return carry
      lax.fori_loop(0, N_CHUNKS, drain, 0)
    plsc.subcore_barrier()

    pltpu.sync_copy(acc.at[pl.ds(s * RPT, RPT)],
                    agg_out.at[c].at[pl.ds(s * RPT, RPT)])
    @pl.when(s == NS - 1)
    def _():
      pltpu.sync_copy(acc.at[pl.ds(NS * RPT, TAIL)],
                      agg_out.at[c].at[pl.ds(NS * RPT, TAIL)])
    if compute_deg:
      @pl.when(s == 0)
      def _():
        pltpu.sync_copy(deg_acc, deg_out.at[c])

  return sc_agg


_sc_agg_deg = _make_sc_agg(True)
_sc_agg = _make_sc_agg(False)


BLK = 1000


def _tc_layer1(x_ref, agg_ref, deg_ref, wa_ref, ba_ref, wb_ref, bb_ref, o_ref):
  deg = deg_ref[0] + deg_ref[1]
  agg = agg_ref[0] + agg_ref[1]
  h = x_ref[...] + agg * (1.0 / jnp.maximum(deg, 1.0))
  t = jnp.maximum(jnp.dot(h, wa_ref[...], preferred_element_type=jnp.float32)
                  + ba_ref[...], 0.0)
  y = jnp.dot(t, wb_ref[...], preferred_element_type=jnp.float32) + bb_ref[...]
  o_ref[...] = jnp.maximum(y, 0.0)


def _tc_layer23(x_ref, agg_ref, deg_ref, w2a_ref, b2a_ref, w2b_ref, b2b_ref,
                w3a_ref, b3a_ref, w3b_ref, b3b_ref, o_ref):
  deg = deg_ref[0] + deg_ref[1]
  agg = agg_ref[0] + agg_ref[1]
  h = x_ref[...] + agg * (1.0 / jnp.maximum(deg, 1.0))
  t = jnp.maximum(jnp.dot(h, w2a_ref[...], preferred_element_type=jnp.float32)
                  + b2a_ref[...], 0.0)
  x2 = jnp.maximum(jnp.dot(t, w2b_ref[...], preferred_element_type=jnp.float32)
                   + b2b_ref[...], 0.0)
  t3 = jnp.maximum(jnp.dot(x2, w3a_ref[...], preferred_element_type=jnp.float32)
                   + b3a_ref[...], 0.0)
  o_ref[...] = (jnp.dot(t3, w3b_ref[...], preferred_element_type=jnp.float32)
                + b3b_ref[...])


def _row_spec():
  return pl.BlockSpec((BLK, D), lambda i: (i, 0))


def _agg_spec():
  return pl.BlockSpec((NC, BLK, D), lambda i: (0, i, 0))


def _deg_spec():
  return pl.BlockSpec((NC, BLK, 1), lambda i: (0, i, 0))


def _w_spec():
  return pl.BlockSpec((D, D), lambda i: (0, 0))


def _b_spec():
  return pl.BlockSpec((1, D), lambda i: (0, 0))


def kernel(features, edge_index, W1a, b1a, W1b, b1b, W2a, b2a, W2b, b2b,
           W3a, b3a, W3b, b3b):
  src = edge_index[0].reshape(NW, NBLK, KBLK, CHUNK)
  dst = edge_index[1].reshape(NW, NBLK, KBLK, CHUNK)
  zrow = jnp.zeros((N, D), jnp.float32)
  zdeg = jnp.zeros((N,), jnp.float32)
  ones = jnp.ones((CHUNK,), jnp.float32)

  aggp1, degp = _sc_agg_deg(features, src, dst, zrow, zdeg, ones)
  degp3 = degp.reshape(NC, N, 1)

  x1 = pl.pallas_call(
      _tc_layer1,
      grid=(N // BLK,),
      in_specs=[_row_spec(), _agg_spec(), _deg_spec(),
                _w_spec(), _b_spec(), _w_spec(), _b_spec()],
      out_specs=_row_spec(),
      out_shape=jax.ShapeDtypeStruct((N, D), jnp.float32),
  )(features, aggp1, degp3, W1a, b1a.reshape(1, D), W1b, b1b.reshape(1, D))

  (aggp2,) = _sc_agg(x1, src, dst, zrow, zdeg, ones)

  out = pl.pallas_call(
      _tc_layer23,
      grid=(N // BLK,),
      in_specs=[_row_spec(), _agg_spec(), _deg_spec(),
                _w_spec(), _b_spec(), _w_spec(), _b_spec(),
                _w_spec(), _b_spec(), _w_spec(), _b_spec()],
      out_specs=_row_spec(),
      out_shape=jax.ShapeDtypeStruct((N, D), jnp.float32),
  )(x1, aggp2, degp3, W2a, b2a.reshape(1, D), W2b, b2b.reshape(1, D),
    W3a, b3a.reshape(1, D), W3b, b3b.reshape(1, D))
  return out


# speedup vs baseline: 1.2749x; 1.1048x over previous
import functools

import jax
import jax.numpy as jnp
from jax import lax
from jax.experimental import pallas as pl
from jax.experimental.pallas import tpu as pltpu
from jax.experimental.pallas import tpu_sc as plsc

N = 10000
E = 320000
D = 128

NC = 2
NS = 16
NW = NC * NS
E_PER_W = E // NW
CHUNK = 50
N_CHUNKS = E_PER_W // CHUNK
KBLK = 20
NBLK = N_CHUNKS // KBLK
NB = 5
LOOK = 4
RPT = 624
TAIL = N - NS * RPT


def _make_sc_agg(compute_deg: bool):
  mesh = plsc.VectorSubcoreMesh(core_axis_name="c", subcore_axis_name="s")
  out_type = [jax.ShapeDtypeStruct((NC, N, D), jnp.float32)]
  if compute_deg:
    out_type.append(jax.ShapeDtypeStruct((NC, N), jnp.float32))

  @functools.partial(
      pl.kernel,
      mesh=mesh,
      out_type=tuple(out_type),
      scratch_types=(
          [pltpu.VMEM((KBLK, CHUNK), jnp.int32) for _ in range(4)]
          + [pltpu.VMEM((CHUNK, D), jnp.float32) for _ in range(NB)]
          + [
              pltpu.VMEM((CHUNK,), jnp.float32),
              pltpu.VMEM_SHARED((N, D), jnp.float32),
              pltpu.VMEM_SHARED((N,), jnp.float32),
          ]
          + [pltpu.SemaphoreType.DMA for _ in range(2 * NB + 2)]
      ),
  )
  def sc_agg(x_hbm, edge_hbm, zdeg_hbm, ones_hbm, *rest):
    if compute_deg:
      agg_out, deg_out = rest[0], rest[1]
      scratch = rest[2:]
    else:
      agg_out = rest[0]
      scratch = rest[1:]
    srcA, dstA, srcB, dstB = scratch[0:4]
    rows = scratch[4:4 + NB]
    ones_v, acc, deg_acc = scratch[4 + NB:7 + NB]
    gsem = scratch[7 + NB:7 + 2 * NB]
    ssem = scratch[7 + 2 * NB:7 + 3 * NB]
    dsem, isem = scratch[7 + 3 * NB:9 + 3 * NB]
    idxbufs = ((srcA, dstA), (srcB, dstB))

    c = lax.axis_index("c")
    s = lax.axis_index("s")
    wid = c * NS + s

    z16 = jnp.zeros((16,), jnp.float32)
    def zrow_body(r, carry):
      for col in range(D // 16):
        rows[0][r, pl.ds(col * 16, 16)] = z16
      return carry
    lax.fori_loop(0, CHUNK, zrow_body, 0)
    nz = RPT // CHUNK
    for k in range(nz):
      pltpu.async_copy(rows[0], acc.at[pl.ds(s * RPT + k * CHUNK, CHUNK)],
                       ssem[0])
    pltpu.async_copy(rows[0].at[pl.ds(0, RPT - nz * CHUNK)],
                     acc.at[pl.ds(s * RPT + nz * CHUNK, RPT - nz * CHUNK)],
                     ssem[0])
    @pl.when(s == NS - 1)
    def _():
      pltpu.async_copy(rows[0].at[pl.ds(0, TAIL)],
                       acc.at[pl.ds(NS * RPT, TAIL)], ssem[0])
    for k in range(nz):
      pltpu.make_async_copy(rows[0], acc.at[pl.ds(0, CHUNK)], ssem[0]).wait()
    pltpu.make_async_copy(rows[0].at[pl.ds(0, RPT - nz * CHUNK)],
                          acc.at[pl.ds(0, RPT - nz * CHUNK)], ssem[0]).wait()
    @pl.when(s == NS - 1)
    def _():
      pltpu.make_async_copy(rows[0].at[pl.ds(0, TAIL)],
                            acc.at[pl.ds(0, TAIL)], ssem[0]).wait()
    if compute_deg:
      @pl.when(s == 0)
      def _():
        pltpu.sync_copy(zdeg_hbm, deg_acc)
      pltpu.sync_copy(ones_hbm, ones_v)
    plsc.subcore_barrier()

    def visit(i, b, src_v, dst_v, skip_drain, guard_issue):
      t = (b + LOOK) % NB
      pltpu.make_async_copy(x_hbm.at[src_v.at[i]], rows[b], gsem[b]).wait()
      if compute_deg:
        pltpu.async_copy(ones_v, deg_acc.at[dst_v.at[i]], dsem, add=True)
      pltpu.async_copy(rows[b], acc.at[dst_v.at[i]], ssem[b], add=True)
      if not skip_drain:
        pltpu.make_async_copy(rows[t], acc.at[dst_v.at[0]], ssem[t]).wait()
      if guard_issue:
        @pl.when(i + LOOK < KBLK)
        def _():
          pltpu.async_copy(x_hbm.at[src_v.at[i + LOOK]], rows[t], gsem[t])
      else:
        pltpu.async_copy(x_hbm.at[src_v.at[i + LOOK]], rows[t], gsem[t])

    for j in range(NBLK):
      src_v, dst_v = idxbufs[j % 2]
      if j == 0:
        pltpu.sync_copy(edge_hbm.at[0].at[wid].at[0], src_v)
        pltpu.sync_copy(edge_hbm.at[1].at[wid].at[0], dst_v)
      else:
        pltpu.make_async_copy(edge_hbm.at[0].at[wid].at[0], src_v, isem).wait()
        pltpu.make_async_copy(edge_hbm.at[1].at[wid].at[0], dst_v, isem).wait()
      for i in range(LOOK):
        pltpu.async_copy(x_hbm.at[src_v.at[i]], rows[i], gsem[i])

      for i in range(NB):
        visit(i, i, src_v, dst_v,
              skip_drain=(j == 0 and i < NB - LOOK), guard_issue=False)

      if j + 1 < NBLK:
        nsrc, ndst = idxbufs[(j + 1) % 2]
        pltpu.async_copy(edge_hbm.at[0].at[wid].at[j + 1], nsrc, isem)
        pltpu.async_copy(edge_hbm.at[1].at[wid].at[j + 1], ndst, isem)

      def body(o, carry2):
        for b4 in range(NB):
          visit(o * NB + b4, b4, src_v, dst_v,
                skip_drain=False, guard_issue=(b4 >= NB - LOOK))
        return carry2

      lax.fori_loop(1, KBLK // NB, body, 0)

    src_v, dst_v = idxbufs[(NBLK - 1) % 2]
    for i in range(KBLK - (NB - LOOK), KBLK):
      pltpu.make_async_copy(rows[i % NB], acc.at[dst_v.at[0]], ssem[i % NB]).wait()
    if compute_deg:
      def drain(i, carry):
        pltpu.make_async_copy(ones_v, deg_acc.at[dst_v.at[0]], dsem).wait()
        return carry
      lax.fori_loop(0, N_CHUNKS, drain, 0)
    plsc.subcore_barrier()

    pltpu.sync_copy(acc.at[pl.ds(s * RPT, RPT)],
                    agg_out.at[c].at[pl.ds(s * RPT, RPT)])
    @pl.when(s == NS - 1)
    def _():
      pltpu.sync_copy(acc.at[pl.ds(NS * RPT, TAIL)],
                      agg_out.at[c].at[pl.ds(NS * RPT, TAIL)])
    if compute_deg:
      @pl.when(s == 0)
      def _():
        pltpu.sync_copy(deg_acc, deg_out.at[c])

  return sc_agg


_sc_agg_deg = _make_sc_agg(True)
_sc_agg = _make_sc_agg(False)


BLK = 1000


def _tc_layer1(x_ref, agg_ref, deg_ref, wa_ref, ba_ref, wb_ref, bb_ref, o_ref):
  deg = (deg_ref[0, 0, 0] + deg_ref[1, 0, 0])[:, None]
  agg = agg_ref[0] + agg_ref[1]
  h = x_ref[...] + agg * (1.0 / jnp.maximum(deg, 1.0))
  t = jnp.maximum(jnp.dot(h, wa_ref[...], preferred_element_type=jnp.float32)
                  + ba_ref[...], 0.0)
  y = jnp.dot(t, wb_ref[...], preferred_element_type=jnp.float32) + bb_ref[...]
  o_ref[...] = jnp.maximum(y, 0.0)


def _tc_layer23(x_ref, agg_ref, deg_ref, w2a_ref, b2a_ref, w2b_ref, b2b_ref,
                w3a_ref, b3a_ref, w3b_ref, b3b_ref, o_ref):
  deg = (deg_ref[0, 0, 0] + deg_ref[1, 0, 0])[:, None]
  agg = agg_ref[0] + agg_ref[1]
  h = x_ref[...] + agg * (1.0 / jnp.maximum(deg, 1.0))
  t = jnp.maximum(jnp.dot(h, w2a_ref[...], preferred_element_type=jnp.float32)
                  + b2a_ref[...], 0.0)
  x2 = jnp.maximum(jnp.dot(t, w2b_ref[...], preferred_element_type=jnp.float32)
                   + b2b_ref[...], 0.0)
  t3 = jnp.maximum(jnp.dot(x2, w3a_ref[...], preferred_element_type=jnp.float32)
                   + b3a_ref[...], 0.0)
  o_ref[...] = (jnp.dot(t3, w3b_ref[...], preferred_element_type=jnp.float32)
                + b3b_ref[...])


def _row_spec():
  return pl.BlockSpec((BLK, D), lambda i: (i, 0))


def _agg_spec():
  return pl.BlockSpec((NC, BLK, D), lambda i: (0, i, 0))


def _deg_spec():
  return pl.BlockSpec((NC, 1, 1, BLK), lambda i: (0, i, 0, 0))


def _w_spec():
  return pl.BlockSpec((D, D), lambda i: (0, 0))


def _b_spec():
  return pl.BlockSpec((1, D), lambda i: (0, 0))


def kernel(features, edge_index, W1a, b1a, W1b, b1b, W2a, b2a, W2b, b2b,
           W3a, b3a, W3b, b3b):
  edges = edge_index.reshape(2, NW, NBLK, KBLK, CHUNK)
  zdeg = jnp.zeros((N,), jnp.float32)
  ones = jnp.ones((CHUNK,), jnp.float32)

  aggp1, degp = _sc_agg_deg(features, edges, zdeg, ones)
  degp3 = degp.reshape(NC, N // BLK, 1, BLK)

  x1 = pl.pallas_call(
      _tc_layer1,
      grid=(N // BLK,),
      in_specs=[_row_spec(), _agg_spec(), _deg_spec(),
                _w_spec(), _b_spec(), _w_spec(), _b_spec()],
      out_specs=_row_spec(),
      out_shape=jax.ShapeDtypeStruct((N, D), jnp.float32),
  )(features, aggp1, degp3, W1a, b1a.reshape(1, D), W1b, b1b.reshape(1, D))

  (aggp2,) = _sc_agg(x1, edges, zdeg, ones)

  out = pl.pallas_call(
      _tc_layer23,
      grid=(N // BLK,),
      in_specs=[_row_spec(), _agg_spec(), _deg_spec(),
                _w_spec(), _b_spec(), _w_spec(), _b_spec(),
                _w_spec(), _b_spec(), _w_spec(), _b_spec()],
      out_specs=_row_spec(),
      out_shape=jax.ShapeDtypeStruct((N, D), jnp.float32),
  )(x1, aggp2, degp3, W2a, b2a.reshape(1, D), W2b, b2b.reshape(1, D),
    W3a, b3a.reshape(1, D), W3b, b3b.reshape(1, D))
  return out
